# baseline (speedup 1.0000x reference)
import jax
import jax.numpy as jnp
from jax import lax
from jax.experimental import pallas as pl
from jax.experimental.pallas import tpu as pltpu, tpu_sc as plsc

EMB = 64
SCALE = 8.0
NC = 2
NS = 16
NW = NC * NS
GS = 4


def _make_sc_embed(n_seq: int, seq_len: int):
    assert n_seq % (NW * GS) == 0
    seq_per_w = n_seq // NW
    n_grp = seq_per_w // GS
    gtok = GS * seq_len
    chunk = 104
    n_chunk, last = divmod(gtok, chunk)
    chunks = [chunk] * n_chunk + ([last] if last else [])

    mesh = plsc.VectorSubcoreMesh(
        core_axis_name="c", subcore_axis_name="s",
        num_cores=NC, num_subcores=NS,
    )

    @pl.kernel(
        out_type=jax.ShapeDtypeStruct((n_seq * seq_len, 2 * EMB), jnp.float32),
        mesh=mesh,
        scratch_types=[
            pltpu.VMEM((2, gtok), jnp.int32),
            pltpu.VMEM((2, gtok, EMB), jnp.float32),
            pltpu.VMEM((seq_len, EMB), jnp.float32),
            pltpu.SemaphoreType.DMA((2,)),
            pltpu.SemaphoreType.DMA((2,)),
            pltpu.SemaphoreType.DMA((2,)),
        ],
        compiler_params=pltpu.CompilerParams(use_tc_tiling_on_sc=False),
    )
    def sc_embed(tok_hbm, pe_hbm, emb_hbm, out_hbm, idx_v, rows_v, pe_v,
                 sem_i, sem_g, sem_o):
        wid = lax.axis_index("s") * NC + lax.axis_index("c")
        tok0 = wid * seq_per_w * seq_len
        pltpu.sync_copy(pe_hbm, pe_v)

        def idx_start(g, b):
            pltpu.async_copy(tok_hbm.at[pl.ds(tok0 + g * gtok, gtok)],
                             idx_v.at[b], sem_i.at[b])

        def idx_wait(b):
            pltpu.make_async_copy(tok_hbm.at[pl.ds(0, gtok)], idx_v.at[b],
                                  sem_i.at[b]).wait()

        def gather_start(b):
            off = 0
            for c in chunks:
                pltpu.async_copy(emb_hbm.at[idx_v.at[b, pl.ds(off, c)]],
                                 rows_v.at[b, pl.ds(off, c)], sem_g.at[b])
                off += c

        def gather_wait(b):
            pltpu.make_async_copy(out_hbm.at[pl.ds(0, gtok), pl.ds(0, EMB)],
                                  rows_v.at[b], sem_g.at[b]).wait()

        def out_start(g, b):
            pltpu.async_copy(rows_v.at[b],
                             out_hbm.at[pl.ds(tok0 + g * gtok, gtok),
                                        pl.ds(0, EMB)],
                             sem_o.at[b])

        def out_wait(b):
            pltpu.make_async_copy(rows_v.at[b],
                                  out_hbm.at[pl.ds(0, gtok), pl.ds(0, EMB)],
                                  sem_o.at[b]).wait()

        def compute(b):
            @pl.loop(0, seq_len)
            def _tok(t):
                for s in range(GS):
                    r = s * seq_len + t
                    for c in range(EMB // 16):
                        sl = pl.ds(c * 16, 16)
                        plsc.addupdate(rows_v.at[b, r, sl], pe_v[t, sl])

        def step(g, b, *, first=False, last=False, stage_idx=True):
            gather_wait(b)
            if not last:
                if stage_idx:
                    idx_start(g + 2, b)
                if not first:
                    out_wait(1 - b)
                idx_wait(1 - b)
                gather_start(1 - b)
            compute(b)
            out_start(g, b)

        idx_start(0, 0)
        idx_wait(0)
        gather_start(0)
        idx_start(1, 1)

        step(0, 0, first=True)

        @pl.loop(0, (n_grp - 4) // 2)
        def _pair(p):
            step(2 * p + 1, 1)
            step(2 * p + 2, 0)

        step(n_grp - 3, 1)
        step(n_grp - 2, 0, stage_idx=False)
        step(n_grp - 1, 1, last=True)
        out_wait(0)
        out_wait(1)

    return sc_embed


def kernel(token_sequences, embedding, positional_embedding):
    n_seq, seq_len = token_sequences.shape
    tok = token_sequences.reshape(-1).astype(jnp.int32)
    pe = positional_embedding[0, :seq_len, :]
    f = _make_sc_embed(n_seq, seq_len)
    emb8 = embedding * jnp.float32(SCALE)
    out = f(tok, pe, emb8)
    return out[:, :EMB].reshape(n_seq, seq_len, EMB)


# speedup vs baseline: 1.1559x; 1.1559x over previous
import jax
import jax.numpy as jnp
from jax import lax
from jax.experimental import pallas as pl
from jax.experimental.pallas import tpu as pltpu, tpu_sc as plsc

EMB = 64
SCALE = 8.0
NC = 2
NS = 16
NW = NC * NS
GS = 4


def _make_sc_embed(n_seq: int, seq_len: int):
    assert n_seq % (NW * GS) == 0
    seq_per_w = n_seq // NW
    n_grp = seq_per_w // GS
    gtok = GS * seq_len
    chunk = 104
    n_chunk, last = divmod(gtok, chunk)
    chunks = [chunk] * n_chunk + ([last] if last else [])

    mesh = plsc.VectorSubcoreMesh(
        core_axis_name="c", subcore_axis_name="s",
        num_cores=NC, num_subcores=NS,
    )

    @pl.kernel(
        out_type=jax.ShapeDtypeStruct((n_seq * seq_len, 2 * EMB), jnp.float32),
        mesh=mesh,
        scratch_types=[
            pltpu.VMEM((2, gtok), jnp.int32),
            pltpu.VMEM((2, gtok, EMB), jnp.float32),
            pltpu.VMEM((seq_len, EMB), jnp.float32),
            pltpu.SemaphoreType.DMA((2,)),
            pltpu.SemaphoreType.DMA((2,)),
            pltpu.SemaphoreType.DMA((2,)),
        ],
        compiler_params=pltpu.CompilerParams(use_tc_tiling_on_sc=False),
    )
    def sc_embed(tok_hbm, pe_hbm, emb_hbm, out_hbm, idx_v, rows_v, pe_v,
                 sem_i, sem_g, sem_o):
        wid = lax.axis_index("s") * NC + lax.axis_index("c")
        tok0 = wid * seq_per_w * seq_len
        pltpu.sync_copy(pe_hbm, pe_v)

        def idx_start(g, b):
            pltpu.async_copy(tok_hbm.at[pl.ds(tok0 + g * gtok, gtok)],
                             idx_v.at[b], sem_i.at[b])

        def idx_wait(b):
            pltpu.make_async_copy(tok_hbm.at[pl.ds(0, gtok)], idx_v.at[b],
                                  sem_i.at[b]).wait()

        def gather_start(b):
            off = 0
            for c in chunks:
                pltpu.async_copy(emb_hbm.at[idx_v.at[b, pl.ds(off, c)]],
                                 rows_v.at[b, pl.ds(off, c)], sem_g.at[b])
                off += c

        def gather_wait(b):
            pltpu.make_async_copy(out_hbm.at[pl.ds(0, gtok), pl.ds(0, EMB)],
                                  rows_v.at[b], sem_g.at[b]).wait()

        def out_start(g, b):
            pltpu.async_copy(rows_v.at[b],
                             out_hbm.at[pl.ds(tok0 + g * gtok, gtok),
                                        pl.ds(0, EMB)],
                             sem_o.at[b])

        def out_wait(b):
            pltpu.make_async_copy(rows_v.at[b],
                                  out_hbm.at[pl.ds(0, gtok), pl.ds(0, EMB)],
                                  sem_o.at[b]).wait()

        def compute(b):
            @pl.loop(0, seq_len, unroll=2)
            def _tok(t):
                pes = [pe_v[t, pl.ds(c * 16, 16)] for c in range(EMB // 16)]
                for s in range(GS):
                    r = s * seq_len + t
                    for c in range(EMB // 16):
                        sl = pl.ds(c * 16, 16)
                        rows_v[b, r, sl] = rows_v[b, r, sl] * SCALE + pes[c]

        def step(g, b, *, first=False, last=False, stage_idx=True):
            gather_wait(b)
            if not last:
                if stage_idx:
                    idx_start(g + 2, b)
                if not first:
                    out_wait(1 - b)
                idx_wait(1 - b)
                gather_start(1 - b)
            compute(b)
            out_start(g, b)

        idx_start(0, 0)
        idx_wait(0)
        gather_start(0)
        idx_start(1, 1)

        step(0, 0, first=True)

        @pl.loop(0, (n_grp - 4) // 2)
        def _pair(p):
            step(2 * p + 1, 1)
            step(2 * p + 2, 0)

        step(n_grp - 3, 1)
        step(n_grp - 2, 0, stage_idx=False)
        step(n_grp - 1, 1, last=True)
        out_wait(0)
        out_wait(1)

    return sc_embed


def kernel(token_sequences, embedding, positional_embedding):
    n_seq, seq_len = token_sequences.shape
    tok = token_sequences.reshape(-1).astype(jnp.int32)
    pe = positional_embedding[0, :seq_len, :]
    f = _make_sc_embed(n_seq, seq_len)
    out = f(tok, pe, embedding)
    return out[:, :EMB].reshape(n_seq, seq_len, EMB)
